# baseline (speedup 1.0000x reference)
import functools

import jax
import jax.numpy as jnp
from jax import lax
from jax.experimental import pallas as pl
from jax.experimental.pallas import tpu as pltpu
from jax.experimental.pallas import tpu_sc as plsc

VOCAB = 100000
HIDDEN = 64
N_LANG = 100
BATCH = 4096
SEQ = 200

LANE = 16
NC, NS = 2, 16
NW = NC * NS
DP = 128
NK = DP // LANE
NKC = 7
UNROLL = 4
U0, U1 = 128, 72

VBLK = 4000
RPW = BATCH // NW
HALF = SEQ // 2


def _scores_tc(embeddings, tw_row, w_t, b_pad):
    def body(emb_ref, tw_ref, wt_ref, b_ref, out_ref):
        y = jnp.dot(
            emb_ref[...], wt_ref[...], preferred_element_type=jnp.float32
        )
        scale = lax.dot_general(
            tw_ref[0],
            jnp.full((1, DP), 1.0, jnp.float32),
            (((0,), (0,)), ((), ())),
            preferred_element_type=jnp.float32,
        )
        out_ref[...] = y * scale + b_ref[...]

    return pl.pallas_call(
        body,
        grid=(VOCAB // VBLK,),
        in_specs=[
            pl.BlockSpec((VBLK, HIDDEN), lambda i: (i, 0)),
            pl.BlockSpec((1, 1, VBLK), lambda i: (i, 0, 0)),
            pl.BlockSpec((HIDDEN, DP), lambda i: (0, 0)),
            pl.BlockSpec((1, DP), lambda i: (0, 0)),
        ],
        out_specs=pl.BlockSpec((VBLK, DP), lambda i: (i, 0)),
        out_shape=jax.ShapeDtypeStruct((VOCAB, DP), jnp.float32),
    )(embeddings, tw_row, w_t, b_pad)


def _gather_max_sc(token_ids, scores):
    mesh = plsc.VectorSubcoreMesh(core_axis_name="c", subcore_axis_name="s")

    @functools.partial(
        pl.kernel,
        out_type=jax.ShapeDtypeStruct((BATCH, 16 * NKC), jnp.float32),
        mesh=mesh,
        scratch_types=[
            pltpu.VMEM((RPW, SEQ), jnp.int32),
            pltpu.VMEM((U0, DP), jnp.float32),
            pltpu.VMEM((U1, DP), jnp.float32),
            pltpu.VMEM((U0, DP), jnp.float32),
            pltpu.VMEM((U1, DP), jnp.float32),
            pltpu.VMEM((RPW, 16 * NKC), jnp.float32),
            pltpu.SemaphoreType.DMA,
            pltpu.SemaphoreType.DMA,
            pltpu.SemaphoreType.DMA,
            pltpu.SemaphoreType.DMA,
        ],
    )
    def run(ids_hbm, table_hbm, out_hbm, ids_v, buf_a, buf_b, buf_c, buf_d,
            out_v, sem_a, sem_b, sem_c, sem_d):
        wid = lax.axis_index("s") * NC + lax.axis_index("c")
        pltpu.sync_copy(ids_hbm.at[pl.ds(wid * RPW, RPW)], ids_v)

        def fire_even(r, buf, sem):
            r = jnp.minimum(r, RPW - 1)
            pltpu.async_copy(
                table_hbm.at[ids_v.at[r, pl.ds(0, U0)]], buf, sem
            )

        def fire_odd(r, buf, sem):
            r = jnp.minimum(r, RPW - 1)
            pltpu.async_copy(
                table_hbm.at[ids_v.at[r, pl.ds(U0, U1)]], buf, sem
            )

        def drain(buf, sem, sl):
            pltpu.make_async_copy(
                table_hbm.at[ids_v.at[0, sl]], buf, sem
            ).wait()

        def reduce_unit(buf, n, acc):
            def step(jj, a):
                j0 = UNROLL * jj
                for u in range(UNROLL):
                    a = tuple(
                        jnp.maximum(
                            a[k], buf[j0 + u, pl.ds(k * LANE, LANE)]
                        )
                        for k in range(NKC)
                    )
                return a

            return lax.fori_loop(0, n // UNROLL, step, acc)

        def fresh():
            return tuple(
                jnp.full((LANE,), -jnp.inf, jnp.float32) for _ in range(NKC)
            )

        def store(r, acc):
            for k in range(NKC):
                out_v[r, pl.ds(k * LANE, LANE)] = acc[k]

        sl0, sl1 = pl.ds(0, U0), pl.ds(U0, U1)
        fire_even(0, buf_a, sem_a)
        fire_odd(0, buf_b, sem_b)
        fire_even(1, buf_c, sem_c)

        def quad_body(p, carry):
            r0 = 2 * p
            fire_odd(r0 + 1, buf_d, sem_d)
            drain(buf_a, sem_a, sl0)
            acc = reduce_unit(buf_a, U0, fresh())
            fire_even(r0 + 2, buf_a, sem_a)
            drain(buf_b, sem_b, sl1)
            acc = reduce_unit(buf_b, U1, acc)
            store(r0, acc)
            fire_odd(r0 + 2, buf_b, sem_b)
            drain(buf_c, sem_c, sl0)
            acc = reduce_unit(buf_c, U0, fresh())
            fire_even(r0 + 3, buf_c, sem_c)
            drain(buf_d, sem_d, sl1)
            acc = reduce_unit(buf_d, U1, acc)
            store(r0 + 1, acc)
            return carry

        lax.fori_loop(0, RPW // 2, quad_body, 0)
        drain(buf_a, sem_a, sl0)
        drain(buf_b, sem_b, sl1)
        drain(buf_c, sem_c, sl0)
        pltpu.sync_copy(out_v, out_hbm.at[pl.ds(wid * RPW, RPW)])

    return run(token_ids, scores)


def kernel(token_ids, embeddings, token_weights, W, b):
    w_t = jnp.zeros((HIDDEN, DP), jnp.float32).at[:, :N_LANG].set(W.T)
    b_pad = jnp.zeros((1, DP), jnp.float32).at[0, :N_LANG].set(b)
    tw_row = token_weights.reshape(VOCAB // VBLK, 1, VBLK)
    scores = _scores_tc(embeddings, tw_row, w_t, b_pad)
    out = _gather_max_sc(token_ids, scores)
    return out[:, :N_LANG]


# speedup vs baseline: 1.1252x; 1.1252x over previous
import functools

import jax
import jax.numpy as jnp
from jax import lax
from jax.experimental import pallas as pl
from jax.experimental.pallas import tpu as pltpu
from jax.experimental.pallas import tpu_sc as plsc

VOCAB = 100000
HIDDEN = 64
N_LANG = 100
BATCH = 4096
SEQ = 200

LANE = 16
NC, NS = 2, 16
NW = NC * NS
DP = 128
NK = DP // LANE
NKC = 7
UNROLL = 4

VBLK = 4000
RPW = BATCH // NW
HALF = SEQ // 2


def _scores_tc(embeddings, tw_row, w_t, b_pad):
    def body(emb_ref, tw_ref, wt_ref, b_ref, out_ref):
        y = jnp.dot(
            emb_ref[...], wt_ref[...], preferred_element_type=jnp.float32
        )
        scale = lax.dot_general(
            tw_ref[0],
            jnp.full((1, DP), 1.0, jnp.float32),
            (((0,), (0,)), ((), ())),
            preferred_element_type=jnp.float32,
        )
        out_ref[...] = y * scale + b_ref[...]

    return pl.pallas_call(
        body,
        grid=(VOCAB // VBLK,),
        in_specs=[
            pl.BlockSpec((VBLK, HIDDEN), lambda i: (i, 0)),
            pl.BlockSpec((1, 1, VBLK), lambda i: (i, 0, 0)),
            pl.BlockSpec((HIDDEN, DP), lambda i: (0, 0)),
            pl.BlockSpec((1, DP), lambda i: (0, 0)),
        ],
        out_specs=pl.BlockSpec((VBLK, DP), lambda i: (i, 0)),
        out_shape=jax.ShapeDtypeStruct((VOCAB, DP), jnp.float32),
    )(embeddings, tw_row, w_t, b_pad)


def _gather_max_sc(ids2, scores):
    mesh = plsc.VectorSubcoreMesh(core_axis_name="c", subcore_axis_name="s")

    @functools.partial(
        pl.kernel,
        out_type=jax.ShapeDtypeStruct((BATCH, 16 * NKC), jnp.float32),
        mesh=mesh,
        scratch_types=[
            pltpu.VMEM((2 * RPW, HALF), jnp.int32),
            pltpu.VMEM((HALF, DP), jnp.float32),
            pltpu.VMEM((HALF, DP), jnp.float32),
            pltpu.VMEM((HALF, DP), jnp.float32),
            pltpu.VMEM((HALF, DP), jnp.float32),
            pltpu.VMEM((HALF, DP), jnp.float32),
            pltpu.VMEM((HALF, DP), jnp.float32),
            pltpu.VMEM((RPW, 16 * NKC), jnp.float32),
            pltpu.SemaphoreType.DMA,
            pltpu.SemaphoreType.DMA,
            pltpu.SemaphoreType.DMA,
            pltpu.SemaphoreType.DMA,
            pltpu.SemaphoreType.DMA,
            pltpu.SemaphoreType.DMA,
        ],
    )
    def run(ids_hbm, table_hbm, out_hbm, ids_v, buf_a, buf_b, buf_c, buf_d,
            buf_e, buf_f, out_v, sem_a, sem_b, sem_c, sem_d, sem_e, sem_f):
        wid = lax.axis_index("s") * NC + lax.axis_index("c")
        pltpu.sync_copy(ids_hbm.at[pl.ds(wid * 2 * RPW, 2 * RPW)], ids_v)
        nunits = 2 * RPW

        def fire(u, buf, sem):
            pltpu.async_copy(
                table_hbm.at[ids_v.at[jnp.minimum(u, nunits - 1)]], buf, sem
            )

        def drain(buf, sem):
            pltpu.make_async_copy(table_hbm.at[ids_v.at[0]], buf, sem).wait()

        def reduce_unit(buf, acc):
            def step(jj, a):
                j0 = UNROLL * jj
                for u in range(UNROLL):
                    a = tuple(
                        jnp.maximum(
                            a[k], buf[j0 + u, pl.ds(k * LANE, LANE)]
                        )
                        for k in range(NKC)
                    )
                return a

            return lax.fori_loop(0, HALF // UNROLL, step, acc)

        def fresh():
            return tuple(
                jnp.full((LANE,), -jnp.inf, jnp.float32) for _ in range(NKC)
            )

        def store(r, acc):
            for k in range(NKC):
                out_v[r, pl.ds(k * LANE, LANE)] = acc[k]

        bufs = (buf_a, buf_b, buf_c, buf_d, buf_e, buf_f)
        sems = (sem_a, sem_b, sem_c, sem_d, sem_e, sem_f)
        for i in range(5):
            fire(i, bufs[i], sems[i])

        def hex_body(p, carry):
            u0 = 6 * p
            acc = fresh()
            for i in range(6):
                fire(u0 + i + 5, bufs[(i + 5) % 6], sems[(i + 5) % 6])
                drain(bufs[i], sems[i])
                acc = reduce_unit(bufs[i], acc)
                if i % 2 == 1:
                    store(3 * p + i // 2, acc)
                    acc = fresh()
            return carry

        lax.fori_loop(0, 42, hex_body, 0)
        acc = fresh()
        for i in range(4):
            drain(bufs[i], sems[i])
            acc = reduce_unit(bufs[i], acc)
            if i % 2 == 1:
                store(126 + i // 2, acc)
                acc = fresh()
        drain(bufs[4], sems[4])
        pltpu.sync_copy(out_v, out_hbm.at[pl.ds(wid * RPW, RPW)])

    return run(ids2, scores)


def kernel(token_ids, embeddings, token_weights, W, b):
    w_t = jnp.zeros((HIDDEN, DP), jnp.float32).at[:, :N_LANG].set(W.T)
    b_pad = jnp.zeros((1, DP), jnp.float32).at[0, :N_LANG].set(b)
    tw_row = token_weights.reshape(VOCAB // VBLK, 1, VBLK)
    scores = _scores_tc(embeddings, tw_row, w_t, b_pad)
    ids2 = token_ids.reshape(2 * BATCH, HALF)
    out = _gather_max_sc(ids2, scores)
    return out[:, :N_LANG]
